# baseline (speedup 1.0000x reference)
import functools

import jax
import jax.numpy as jnp
from jax import lax
from jax.experimental import pallas as pl
from jax.experimental.pallas import tpu as pltpu
from jax.experimental.pallas import tpu_sc as plsc

BATCH = 16384
D_NUM = 13
N_CAT = 26
D_TOKEN = 32
N_TOK = D_NUM + N_CAT

_info = plsc.get_sparse_core_info()
NC, NS, L = _info.num_cores, _info.num_subcores, _info.num_lanes
NW = NC * NS
BPW = BATCH // NW

C = 32
G = BPW // C
R = C * N_CAT
DMA_ROWS = 64
N_DMA = R // DMA_ROWS


def _encoder_body(xnumf_hbm, xcatf_hbm, weight_hbm, table_hbm, bias_hbm,
                  offs_hbm, out_hbm,
                  xcatf_v, xnumf_v, offs_v, weight_v, bias_v, idx_v, rows_v,
                  stage_v, sem):
    wid = lax.axis_index("s") * NC + lax.axis_index("c")

    pltpu.sync_copy(offs_hbm, offs_v)
    pltpu.sync_copy(weight_hbm, weight_v)
    pltpu.sync_copy(bias_hbm, bias_v)

    def chunk_body(g, carry):
        base = wid * BPW + g * C

        pltpu.sync_copy(xcatf_hbm.at[pl.ds(base * N_CAT, R)], xcatf_v)
        pltpu.sync_copy(xnumf_hbm.at[pl.ds(base, C)], xnumf_v)

        for r in range(N_DMA):
            for q in range(DMA_ROWS // L):
                p = r * DMA_ROWS + q * L
                idx_v[r, pl.ds(q * L, L)] = (
                    xcatf_v[pl.ds(p, L)] + offs_v[pl.ds(p, L)])

        handles = [
            pltpu.async_copy(table_hbm.at[idx_v.at[r]],
                             rows_v.at[pl.ds(r * DMA_ROWS, DMA_ROWS)], sem)
            for r in range(N_DMA)
        ]
        for h in handles:
            h.wait()

        wnum = [weight_v[d, pl.ds(h * L, L)] for d in range(D_NUM) for h in range(2)]
        bnum = [bias_v[d, pl.ds(h * L, L)] for d in range(D_NUM) for h in range(2)]

        def num_body(c, carry2):
            row0 = c * N_TOK
            xrow = xnumf_v[c, pl.ds(0, L)]
            for d in range(D_NUM):
                sv = jnp.full((L,), xrow[d], jnp.float32)
                for h in range(2):
                    stage_v[row0 + d, pl.ds(h * L, L)] = (
                        sv * wnum[2 * d + h] + bnum[2 * d + h])
            return carry2

        lax.fori_loop(0, C, num_body, 0)

        bcat = [bias_v[D_NUM + j, pl.ds(h * L, L)]
                for j in range(N_CAT) for h in range(2)]

        def cat_body(c, carry2):
            row0 = c * N_TOK + D_NUM
            src0 = c * N_CAT
            for j in range(N_CAT):
                for h in range(2):
                    stage_v[row0 + j, pl.ds(h * L, L)] = (
                        rows_v[src0 + j, pl.ds(h * L, L)] + bcat[2 * j + h])
            return carry2

        lax.fori_loop(0, C, cat_body, 0)

        pltpu.sync_copy(stage_v, out_hbm.at[pl.ds(base * N_TOK, C * N_TOK)])
        return carry

    lax.fori_loop(0, G, chunk_body, 0)


@jax.jit
def _encoder(x_numf, x_catf, weight, cat_table, tab_bias, offs_pad):
    mesh = plsc.VectorSubcoreMesh(core_axis_name="c", subcore_axis_name="s")
    f = pl.kernel(
        _encoder_body, mesh=mesh,
        compiler_params=pltpu.CompilerParams(use_tc_tiling_on_sc=False),
        out_type=jax.ShapeDtypeStruct((BATCH * N_TOK, D_TOKEN), jnp.float32),

        scratch_types=[
            pltpu.VMEM((R,), jnp.int32),
            pltpu.VMEM((C, L), jnp.float32),
            pltpu.VMEM((R,), jnp.int32),
            pltpu.VMEM((D_NUM, D_TOKEN), jnp.float32),
            pltpu.VMEM((N_TOK, D_TOKEN), jnp.float32),
            pltpu.VMEM((N_DMA, DMA_ROWS), jnp.int32),
            pltpu.VMEM((R, D_TOKEN), jnp.float32),
            pltpu.VMEM((C * N_TOK, D_TOKEN), jnp.float32),
            pltpu.SemaphoreType.DMA,
        ],
    )
    return f(x_numf, x_catf, weight, cat_table, tab_bias, offs_pad)


def kernel(x_num, x_cat, weight, cat_table, tab_bias, category_offsets):
    x_numf = jnp.pad(x_num, ((0, 0), (0, L - D_NUM)))
    x_catf = x_cat.reshape(BATCH * N_CAT)
    offs_pad = jnp.tile(category_offsets.astype(jnp.int32), C)
    out = _encoder(x_numf, x_catf, weight, cat_table, tab_bias, offs_pad)
    return out.reshape(BATCH, N_TOK, D_TOKEN)
